# baseline (speedup 1.0000x reference)
import functools

import jax
import jax.numpy as jnp
from jax import lax
from jax.experimental import pallas as pl
from jax.experimental.pallas import tpu as pltpu
from jax.experimental.pallas import tpu_sc as plsc

N = 10000
E = 320000
C = 128
NC = 2
NS = 16
NW = NC * NS
EPW = E // NW
NPAD = 10240
RPS = NPAD // NS
TRASH = NPAD - 1
EPW2 = 10240
CHUNK = 128
NCHUNK = EPW2 // CHUNK
G = 40
NG = NCHUNK // G

_mesh = plsc.VectorSubcoreMesh(core_axis_name="c", subcore_axis_name="s",
                               num_cores=NC, num_subcores=NS)


def _make_segsum(with_deg: bool):
    out_type = [jax.ShapeDtypeStruct((NC, NPAD, C), jnp.float32)]
    scratch = [
        pltpu.VMEM((G, CHUNK), jnp.int32),
        pltpu.VMEM((G, CHUNK), jnp.int32),
        pltpu.VMEM((CHUNK, C), jnp.float32),
        pltpu.VMEM((CHUNK, C), jnp.float32),
        pltpu.SemaphoreType.DMA,
        pltpu.SemaphoreType.DMA,
        pltpu.SemaphoreType.DMA,
        pltpu.SemaphoreType.DMA,
        pltpu.VMEM_SHARED((NPAD, C), jnp.float32),
    ]
    if with_deg:
        out_type.append(jax.ShapeDtypeStruct((NC, NPAD), jnp.float32))
        scratch += [
            pltpu.VMEM((CHUNK,), jnp.float32),
            pltpu.SemaphoreType.DMA,
            pltpu.VMEM_SHARED((NPAD,), jnp.float32),
        ]

    @functools.partial(pl.kernel, out_type=out_type, mesh=_mesh,
                       scratch_types=scratch)
    def segsum(*refs):
        if with_deg:
            (feat, srcs, dsts, zeros2, zeros1, ones1, aggr_out, deg_out,
             idx_s, idx_d, rows0, rows1, gsem0, gsem1, ssem0, ssem1,
             aggr_sh, ones_v, dsem, deg_sh) = refs
        else:
            (feat, srcs, dsts, zeros2, aggr_out,
             idx_s, idx_d, rows0, rows1, gsem0, gsem1, ssem0, ssem1,
             aggr_sh) = refs
        c = lax.axis_index("c")
        s = lax.axis_index("s")
        wid = s * NC + c
        row0 = s * RPS
        pltpu.sync_copy(zeros2, aggr_sh.at[pl.ds(row0, RPS)])
        if with_deg:
            pltpu.sync_copy(zeros1, deg_sh.at[pl.ds(row0, RPS)])
            pltpu.sync_copy(ones1, ones_v)
        plsc.subcore_barrier()

        def gissue(j, rows, gsem):
            pltpu.async_copy(feat.at[idx_s.at[j]], rows, gsem)

        def gwait(j, rows, gsem):
            pltpu.make_async_copy(feat.at[idx_s.at[j]], rows, gsem).wait()

        def sissue(j, rows, ssem):
            pltpu.async_copy(rows, aggr_sh.at[idx_d.at[j]], ssem, add=True)

        def swait(j, rows, ssem):
            pltpu.make_async_copy(rows, aggr_sh.at[idx_d.at[j]], ssem).wait()

        def dissue(j):
            pltpu.async_copy(ones_v, deg_sh.at[idx_d.at[j]], dsem, add=True)

        def dwait(j):
            pltpu.make_async_copy(ones_v, deg_sh.at[idx_d.at[j]], dsem).wait()

        def group(g, carry):
            pltpu.sync_copy(srcs.at[wid, pl.ds(g * G, G)], idx_s)
            pltpu.sync_copy(dsts.at[wid, pl.ds(g * G, G)], idx_d)

            gissue(0, rows0, gsem0)
            gwait(0, rows0, gsem0)
            sissue(0, rows0, ssem0)
            if with_deg:
                dissue(0)
            gissue(1, rows1, gsem1)

            def body(i, carry2):
                for step, (rows, gsem, ssem, orows, osem) in enumerate(
                        ((rows1, gsem1, ssem1, rows0, ssem0),
                         (rows0, gsem0, ssem0, rows1, ssem1))):
                    j = 2 * i + 1 + step
                    gwait(j, rows, gsem)
                    sissue(j, rows, ssem)
                    if with_deg:
                        dwait(j - 1)
                        dissue(j)
                    swait(j - 1, orows, osem)
                    gissue(j + 1, orows, gsem0 if step == 0 else gsem1)
                return carry2

            lax.fori_loop(0, (G - 2) // 2, body, 0)
            j = G - 1
            gwait(j, rows1, gsem1)
            sissue(j, rows1, ssem1)
            if with_deg:
                dwait(j - 1)
                dissue(j)
            swait(j - 1, rows0, ssem0)
            swait(j, rows1, ssem1)
            if with_deg:
                dwait(j)
            return carry

        lax.fori_loop(0, NG, group, 0)
        plsc.subcore_barrier()
        pltpu.sync_copy(aggr_sh.at[pl.ds(row0, RPS)],
                        aggr_out.at[c, pl.ds(row0, RPS)])
        if with_deg:
            pltpu.sync_copy(deg_sh.at[pl.ds(row0, RPS)],
                            deg_out.at[c, pl.ds(row0, RPS)])

    return segsum


_segsum_deg = _make_segsum(True)
_segsum = _make_segsum(False)

BLK = 1280
GRID = NPAD // BLK


def _t1_body(aggr_ref, deg_ref, x_ref, wl_ref, wr_ref, b_ref, o_ref):
    aggr = aggr_ref[0] + aggr_ref[1]
    deg = deg_ref[0] + deg_ref[1]
    mean = aggr / jnp.maximum(deg, 1.0)
    h = jnp.dot(mean, wl_ref[...], preferred_element_type=jnp.float32)
    h += jnp.dot(x_ref[...], wr_ref[...], preferred_element_type=jnp.float32)
    h += b_ref[...]
    o_ref[...] = jnp.maximum(h, 0.0)


def _t2_body(aggr_ref, deg_ref, h_ref, wl_ref, wr_ref, b_ref,
             wlin_ref, blin_ref, o_ref):
    aggr = aggr_ref[0] + aggr_ref[1]
    deg = deg_ref[0] + deg_ref[1]
    mean = aggr / jnp.maximum(deg, 1.0)
    h2 = jnp.dot(mean, wl_ref[...], preferred_element_type=jnp.float32)
    h2 += jnp.dot(h_ref[...], wr_ref[...], preferred_element_type=jnp.float32)
    h2 += b_ref[...]
    h2 = jnp.maximum(h2, 0.0)
    o_ref[...] = (jnp.dot(h2, wlin_ref[...], preferred_element_type=jnp.float32)
                  + blin_ref[...])


_W_SPEC = pl.BlockSpec((C, C), lambda i: (0, 0))
_B_SPEC = pl.BlockSpec((1, C), lambda i: (0, 0))
_ROW_SPEC = pl.BlockSpec((BLK, C), lambda i: (i, 0))
_AGGR_SPEC = pl.BlockSpec((NC, BLK, C), lambda i: (0, i, 0))
_DEG_SPEC = pl.BlockSpec((NC, BLK, 1), lambda i: (0, i, 0))

_t1 = pl.pallas_call(
    _t1_body,
    grid=(GRID,),
    in_specs=[_AGGR_SPEC, _DEG_SPEC, _ROW_SPEC, _W_SPEC, _W_SPEC, _B_SPEC],
    out_specs=_ROW_SPEC,
    out_shape=jax.ShapeDtypeStruct((NPAD, C), jnp.float32),
)

_t2 = pl.pallas_call(
    _t2_body,
    grid=(GRID,),
    in_specs=[_AGGR_SPEC, _DEG_SPEC, _ROW_SPEC, _W_SPEC, _W_SPEC, _B_SPEC,
              _W_SPEC, _B_SPEC],
    out_specs=_ROW_SPEC,
    out_shape=jax.ShapeDtypeStruct((NPAD, C), jnp.float32),
)


def kernel(x, edge_index, Wl1, Wr1, b1, Wl2, Wr2, b2, Wlin, blin):
    npad_e = EPW2 - EPW
    src = jnp.concatenate(
        [edge_index[0].astype(jnp.int32).reshape(NW, EPW),
         jnp.broadcast_to(jnp.arange(npad_e, dtype=jnp.int32), (NW, npad_e))],
        axis=1).reshape(NW, NCHUNK, CHUNK)
    dst = jnp.concatenate(
        [edge_index[1].astype(jnp.int32).reshape(NW, EPW),
         jnp.broadcast_to(jnp.arange(N, N + npad_e, dtype=jnp.int32),
                          (NW, npad_e))],
        axis=1).reshape(NW, NCHUNK, CHUNK)
    xp = jnp.pad(x, ((0, NPAD - N), (0, 0)))
    zeros2 = jnp.zeros((RPS, C), jnp.float32)
    zeros1 = jnp.zeros((RPS,), jnp.float32)
    ones1 = jnp.ones((CHUNK,), jnp.float32)

    aggr1, deg = _segsum_deg(xp, src, dst, zeros2, zeros1, ones1)
    deg3 = deg.reshape(NC, NPAD, 1)
    h = _t1(aggr1, deg3, xp, Wl1, Wr1, b1.reshape(1, C))
    (aggr2,) = _segsum(h, src, dst, zeros2)
    out = _t2(aggr2, deg3, h, Wl2, Wr2, b2.reshape(1, C),
              Wlin, blin.reshape(1, C))
    return out[:N]


# speedup vs baseline: 1.1423x; 1.1423x over previous
import functools

import jax
import jax.numpy as jnp
from jax import lax
from jax.experimental import pallas as pl
from jax.experimental.pallas import tpu as pltpu
from jax.experimental.pallas import tpu_sc as plsc

N = 10000
E = 320000
C = 128
NC = 2
NS = 16
NW = NC * NS
EPW = E // NW
NPAD = 10240
RPS = NPAD // NS
TRASH = NPAD - 1
EPW2 = 10240
CHUNK = 128
NCHUNK = EPW2 // CHUNK
G = 40
NG = NCHUNK // G

_mesh = plsc.VectorSubcoreMesh(core_axis_name="c", subcore_axis_name="s",
                               num_cores=NC, num_subcores=NS)


def _make_segsum(with_deg: bool):
    out_type = [jax.ShapeDtypeStruct((NC, NPAD, C), jnp.float32)]
    scratch = [
        pltpu.VMEM((G, CHUNK), jnp.int32),
        pltpu.VMEM((G, CHUNK), jnp.int32),
        pltpu.VMEM((CHUNK, C), jnp.float32),
        pltpu.VMEM((CHUNK, C), jnp.float32),
        pltpu.SemaphoreType.DMA,
        pltpu.SemaphoreType.DMA,
        pltpu.SemaphoreType.DMA,
        pltpu.SemaphoreType.DMA,
        pltpu.VMEM_SHARED((NPAD, C), jnp.float32),
    ]
    if with_deg:
        out_type.append(jax.ShapeDtypeStruct((NC, NPAD), jnp.float32))
        scratch += [
            pltpu.VMEM((CHUNK,), jnp.float32),
            pltpu.SemaphoreType.DMA,
            pltpu.VMEM_SHARED((NPAD,), jnp.float32),
        ]

    @functools.partial(pl.kernel, out_type=out_type, mesh=_mesh,
                       scratch_types=scratch)
    def segsum(*refs):
        if with_deg:
            (feat, srcs, dsts, zeros2, zeros1, ones1, aggr_out, deg_out,
             idx_s, idx_d, rows0, rows1, gsem0, gsem1, ssem0, ssem1,
             aggr_sh, ones_v, dsem, deg_sh) = refs
        else:
            (feat, srcs, dsts, zeros2, aggr_out,
             idx_s, idx_d, rows0, rows1, gsem0, gsem1, ssem0, ssem1,
             aggr_sh) = refs
        c = lax.axis_index("c")
        s = lax.axis_index("s")
        wid = s * NC + c
        row0 = s * RPS
        pltpu.sync_copy(zeros2, aggr_sh.at[pl.ds(row0, RPS)])
        if with_deg:
            pltpu.sync_copy(zeros1, deg_sh.at[pl.ds(row0, RPS)])
            pltpu.sync_copy(ones1, ones_v)
        plsc.subcore_barrier()

        def gissue(j, rows, gsem):
            pltpu.async_copy(feat.at[idx_s.at[j]], rows, gsem)

        def gwait(j, rows, gsem):
            pltpu.make_async_copy(feat.at[idx_s.at[j]], rows, gsem).wait()

        def sissue(j, rows, ssem):
            pltpu.async_copy(rows, aggr_sh.at[idx_d.at[j]], ssem, add=True)

        def swait(j, rows, ssem):
            pltpu.make_async_copy(rows, aggr_sh.at[idx_d.at[j]], ssem).wait()

        def dissue(j):
            pltpu.async_copy(ones_v, deg_sh.at[idx_d.at[j]], dsem, add=True)

        def dwait(j):
            pltpu.make_async_copy(ones_v, deg_sh.at[idx_d.at[j]], dsem).wait()

        def group(g, carry):
            pltpu.sync_copy(srcs.at[wid, pl.ds(g * G, G)], idx_s)
            pltpu.sync_copy(dsts.at[wid, pl.ds(g * G, G)], idx_d)
            gissue(0, rows0, gsem0)
            gissue(1, rows1, gsem1)
            if with_deg:
                dissue(0)

            def body(i, carry2):
                jj = 2 * i
                for b, (rows, gsem) in enumerate(((rows0, gsem0),
                                                  (rows1, gsem1))):
                    j = jj + b
                    gwait(j, rows, gsem)
                    pltpu.sync_copy(rows, aggr_sh.at[idx_d.at[j]], add=True)
                    if with_deg:
                        dwait(j)
                        dissue(j + 1)
                    gissue(j + 2, rows, gsem)
                return carry2

            lax.fori_loop(0, G // 2 - 1, body, 0)
            for b, (rows, gsem) in enumerate(((rows0, gsem0),
                                              (rows1, gsem1))):
                j = G - 2 + b
                gwait(j, rows, gsem)
                pltpu.sync_copy(rows, aggr_sh.at[idx_d.at[j]], add=True)
                if with_deg:
                    dwait(j)
                    if b == 0:
                        dissue(j + 1)
            return carry

        lax.fori_loop(0, NG, group, 0)
        plsc.subcore_barrier()
        pltpu.sync_copy(aggr_sh.at[pl.ds(row0, RPS)],
                        aggr_out.at[c, pl.ds(row0, RPS)])
        if with_deg:
            pltpu.sync_copy(deg_sh.at[pl.ds(row0, RPS)],
                            deg_out.at[c, pl.ds(row0, RPS)])

    return segsum


_segsum_deg = _make_segsum(True)
_segsum = _make_segsum(False)

BLK = 1280
GRID = NPAD // BLK


def _t1_body(aggr_ref, deg_ref, x_ref, wl_ref, wr_ref, b_ref, o_ref):
    aggr = aggr_ref[0] + aggr_ref[1]
    deg = deg_ref[0] + deg_ref[1]
    mean = aggr / jnp.maximum(deg, 1.0)
    h = jnp.dot(mean, wl_ref[...], preferred_element_type=jnp.float32)
    h += jnp.dot(x_ref[...], wr_ref[...], preferred_element_type=jnp.float32)
    h += b_ref[...]
    o_ref[...] = jnp.maximum(h, 0.0)


def _t2_body(aggr_ref, deg_ref, h_ref, wl_ref, wr_ref, b_ref,
             wlin_ref, blin_ref, o_ref):
    aggr = aggr_ref[0] + aggr_ref[1]
    deg = deg_ref[0] + deg_ref[1]
    mean = aggr / jnp.maximum(deg, 1.0)
    h2 = jnp.dot(mean, wl_ref[...], preferred_element_type=jnp.float32)
    h2 += jnp.dot(h_ref[...], wr_ref[...], preferred_element_type=jnp.float32)
    h2 += b_ref[...]
    h2 = jnp.maximum(h2, 0.0)
    o_ref[...] = (jnp.dot(h2, wlin_ref[...], preferred_element_type=jnp.float32)
                  + blin_ref[...])


_W_SPEC = pl.BlockSpec((C, C), lambda i: (0, 0))
_B_SPEC = pl.BlockSpec((1, C), lambda i: (0, 0))
_ROW_SPEC = pl.BlockSpec((BLK, C), lambda i: (i, 0))
_AGGR_SPEC = pl.BlockSpec((NC, BLK, C), lambda i: (0, i, 0))
_DEG_SPEC = pl.BlockSpec((NC, BLK, 1), lambda i: (0, i, 0))

_t1 = pl.pallas_call(
    _t1_body,
    grid=(GRID,),
    in_specs=[_AGGR_SPEC, _DEG_SPEC, _ROW_SPEC, _W_SPEC, _W_SPEC, _B_SPEC],
    out_specs=_ROW_SPEC,
    out_shape=jax.ShapeDtypeStruct((NPAD, C), jnp.float32),
)

_t2 = pl.pallas_call(
    _t2_body,
    grid=(GRID,),
    in_specs=[_AGGR_SPEC, _DEG_SPEC, _ROW_SPEC, _W_SPEC, _W_SPEC, _B_SPEC,
              _W_SPEC, _B_SPEC],
    out_specs=_ROW_SPEC,
    out_shape=jax.ShapeDtypeStruct((NPAD, C), jnp.float32),
)


def kernel(x, edge_index, Wl1, Wr1, b1, Wl2, Wr2, b2, Wlin, blin):
    npad_e = EPW2 - EPW
    src = jnp.concatenate(
        [edge_index[0].astype(jnp.int32).reshape(NW, EPW),
         jnp.broadcast_to(jnp.arange(npad_e, dtype=jnp.int32), (NW, npad_e))],
        axis=1).reshape(NW, NCHUNK, CHUNK)
    dst = jnp.concatenate(
        [edge_index[1].astype(jnp.int32).reshape(NW, EPW),
         jnp.broadcast_to(jnp.arange(N, N + npad_e, dtype=jnp.int32),
                          (NW, npad_e))],
        axis=1).reshape(NW, NCHUNK, CHUNK)
    xp = jnp.pad(x, ((0, NPAD - N), (0, 0)))
    zeros2 = jnp.zeros((RPS, C), jnp.float32)
    zeros1 = jnp.zeros((RPS,), jnp.float32)
    ones1 = jnp.ones((CHUNK,), jnp.float32)

    aggr1, deg = _segsum_deg(xp, src, dst, zeros2, zeros1, ones1)
    deg3 = deg.reshape(NC, NPAD, 1)
    h = _t1(aggr1, deg3, xp, Wl1, Wr1, b1.reshape(1, C))
    (aggr2,) = _segsum(h, src, dst, zeros2)
    out = _t2(aggr2, deg3, h, Wl2, Wr2, b2.reshape(1, C),
              Wlin, blin.reshape(1, C))
    return out[:N]


# speedup vs baseline: 1.1573x; 1.0131x over previous
import functools

import jax
import jax.numpy as jnp
from jax import lax
from jax.experimental import pallas as pl
from jax.experimental.pallas import tpu as pltpu
from jax.experimental.pallas import tpu_sc as plsc

N = 10000
E = 320000
C = 128
NC = 2
NS = 16
NW = NC * NS
EPW = E // NW
NPAD = 10240
RPS = NPAD // NS
TRASH = NPAD - 1
EPW2 = 10240
CHUNK = 128
NCHUNK = EPW2 // CHUNK
G = 40
NG = NCHUNK // G

_mesh = plsc.VectorSubcoreMesh(core_axis_name="c", subcore_axis_name="s",
                               num_cores=NC, num_subcores=NS)


LASTR = N - (NS - 1) * RPS


def _make_segsum(with_deg: bool):
    out_type = [jax.ShapeDtypeStruct((NC, NPAD, C), jnp.float32)]
    scratch = [
        pltpu.VMEM((G, CHUNK), jnp.int32),
        pltpu.VMEM((G, CHUNK), jnp.int32),
        pltpu.VMEM((CHUNK, C), jnp.float32),
        pltpu.VMEM((CHUNK, C), jnp.float32),
        pltpu.SemaphoreType.DMA,
        pltpu.SemaphoreType.DMA,
        pltpu.SemaphoreType.DMA,
        pltpu.SemaphoreType.DMA,
        pltpu.VMEM_SHARED((NPAD, C), jnp.float32),
    ]
    if with_deg:
        out_type.append(jax.ShapeDtypeStruct((NC, NPAD), jnp.float32))
        scratch += [
            pltpu.VMEM((CHUNK,), jnp.float32),
            pltpu.SemaphoreType.DMA,
            pltpu.VMEM_SHARED((NPAD,), jnp.float32),
        ]

    @functools.partial(pl.kernel, out_type=out_type, mesh=_mesh,
                       scratch_types=scratch)
    def segsum(*refs):
        if with_deg:
            (feat, srcs, dsts, zeros2, zeros1, ones1, aggr_out, deg_out,
             idx_s, idx_d, rows0, rows1, gsem0, gsem1, ssem0, ssem1,
             aggr_sh, ones_v, dsem, deg_sh) = refs
        else:
            (feat, srcs, dsts, zeros2, aggr_out,
             idx_s, idx_d, rows0, rows1, gsem0, gsem1, ssem0, ssem1,
             aggr_sh) = refs
        c = lax.axis_index("c")
        s = lax.axis_index("s")
        wid = s * NC + c
        row0 = s * RPS
        pltpu.sync_copy(zeros2, aggr_sh.at[pl.ds(row0, RPS)])
        if with_deg:
            pltpu.sync_copy(zeros1, deg_sh.at[pl.ds(row0, RPS)])
            pltpu.sync_copy(ones1, ones_v)
        plsc.subcore_barrier()

        def gissue(j, rows, gsem):
            pltpu.async_copy(feat.at[idx_s.at[j]], rows, gsem)

        def gwait(j, rows, gsem):
            pltpu.make_async_copy(feat.at[idx_s.at[j]], rows, gsem).wait()

        def sissue(j, rows, ssem):
            pltpu.async_copy(rows, aggr_sh.at[idx_d.at[j]], ssem, add=True)

        def swait(j, rows, ssem):
            pltpu.make_async_copy(rows, aggr_sh.at[idx_d.at[j]], ssem).wait()

        def dissue(j):
            pltpu.async_copy(ones_v, deg_sh.at[idx_d.at[j]], dsem, add=True)

        def dwait(j):
            pltpu.make_async_copy(ones_v, deg_sh.at[idx_d.at[j]], dsem).wait()

        def group(g, carry):
            pltpu.sync_copy(srcs.at[wid, pl.ds(g * G, G)], idx_s)
            pltpu.sync_copy(dsts.at[wid, pl.ds(g * G, G)], idx_d)
            gissue(0, rows0, gsem0)
            gissue(1, rows1, gsem1)
            if with_deg:
                dissue(0)

            def body(i, carry2):
                jj = 2 * i
                for b, (rows, gsem) in enumerate(((rows0, gsem0),
                                                  (rows1, gsem1))):
                    j = jj + b
                    gwait(j, rows, gsem)
                    pltpu.sync_copy(rows, aggr_sh.at[idx_d.at[j]], add=True)
                    if with_deg:
                        dwait(j)
                        dissue(j + 1)
                    gissue(j + 2, rows, gsem)
                return carry2

            lax.fori_loop(0, G // 2 - 1, body, 0)
            for b, (rows, gsem) in enumerate(((rows0, gsem0),
                                              (rows1, gsem1))):
                j = G - 2 + b
                gwait(j, rows, gsem)
                pltpu.sync_copy(rows, aggr_sh.at[idx_d.at[j]], add=True)
                if with_deg:
                    dwait(j)
                    if b == 0:
                        dissue(j + 1)
            return carry

        lax.fori_loop(0, NG, group, 0)
        plsc.subcore_barrier()
        pltpu.sync_copy(aggr_sh.at[pl.ds(row0, RPS)],
                        aggr_out.at[c, pl.ds(row0, RPS)])
        if with_deg:
            pltpu.sync_copy(deg_sh.at[pl.ds(row0, RPS)],
                            deg_out.at[c, pl.ds(row0, RPS)])

    return segsum


_segsum_deg = _make_segsum(True)
_segsum = _make_segsum(False)

BLK = 1000
GRID = N // BLK


def _t1_body(aggr_ref, deg_ref, x_ref, wl_ref, wr_ref, b_ref, o_ref):
    aggr = aggr_ref[0] + aggr_ref[1]
    deg = deg_ref[0] + deg_ref[1]
    mean = aggr / jnp.maximum(deg, 1.0)
    h = jnp.dot(mean, wl_ref[...], preferred_element_type=jnp.float32)
    h += jnp.dot(x_ref[...], wr_ref[...], preferred_element_type=jnp.float32)
    h += b_ref[...]
    o_ref[...] = jnp.maximum(h, 0.0)


def _t2_body(aggr_ref, deg_ref, h_ref, wl_ref, wr_ref, b_ref,
             wlin_ref, blin_ref, o_ref):
    aggr = aggr_ref[0] + aggr_ref[1]
    deg = deg_ref[0] + deg_ref[1]
    mean = aggr / jnp.maximum(deg, 1.0)
    h2 = jnp.dot(mean, wl_ref[...], preferred_element_type=jnp.float32)
    h2 += jnp.dot(h_ref[...], wr_ref[...], preferred_element_type=jnp.float32)
    h2 += b_ref[...]
    h2 = jnp.maximum(h2, 0.0)
    o_ref[...] = (jnp.dot(h2, wlin_ref[...], preferred_element_type=jnp.float32)
                  + blin_ref[...])


_W_SPEC = pl.BlockSpec((C, C), lambda i: (0, 0))
_B_SPEC = pl.BlockSpec((1, C), lambda i: (0, 0))
_ROW_SPEC = pl.BlockSpec((BLK, C), lambda i: (i, 0))
_AGGR_SPEC = pl.BlockSpec((NC, BLK, C), lambda i: (0, i, 0))
_DEG_SPEC = pl.BlockSpec((NC, BLK, 1), lambda i: (0, i, 0))

_t1 = pl.pallas_call(
    _t1_body,
    grid=(GRID,),
    in_specs=[_AGGR_SPEC, _DEG_SPEC, _ROW_SPEC, _W_SPEC, _W_SPEC, _B_SPEC],
    out_specs=_ROW_SPEC,
    out_shape=jax.ShapeDtypeStruct((N, C), jnp.float32),
)

_t2 = pl.pallas_call(
    _t2_body,
    grid=(GRID,),
    in_specs=[_AGGR_SPEC, _DEG_SPEC, _ROW_SPEC, _W_SPEC, _W_SPEC, _B_SPEC,
              _W_SPEC, _B_SPEC],
    out_specs=_ROW_SPEC,
    out_shape=jax.ShapeDtypeStruct((N, C), jnp.float32),
)


def kernel(x, edge_index, Wl1, Wr1, b1, Wl2, Wr2, b2, Wlin, blin):
    npad_e = EPW2 - EPW
    src = jnp.concatenate(
        [edge_index[0].astype(jnp.int32).reshape(NW, EPW),
         jnp.broadcast_to(jnp.arange(npad_e, dtype=jnp.int32), (NW, npad_e))],
        axis=1).reshape(NW, NCHUNK, CHUNK)
    dst = jnp.concatenate(
        [edge_index[1].astype(jnp.int32).reshape(NW, EPW),
         jnp.broadcast_to(jnp.arange(N, N + npad_e, dtype=jnp.int32),
                          (NW, npad_e))],
        axis=1).reshape(NW, NCHUNK, CHUNK)
    zeros2 = jnp.zeros((RPS, C), jnp.float32)
    zeros1 = jnp.zeros((RPS,), jnp.float32)
    ones1 = jnp.ones((CHUNK,), jnp.float32)

    aggr1, deg = _segsum_deg(x, src, dst, zeros2, zeros1, ones1)
    deg3 = deg.reshape(NC, NPAD, 1)
    h = _t1(aggr1, deg3, x, Wl1, Wr1, b1.reshape(1, C))
    (aggr2,) = _segsum(h, src, dst, zeros2)
    return _t2(aggr2, deg3, h, Wl2, Wr2, b2.reshape(1, C),
               Wlin, blin.reshape(1, C))


# speedup vs baseline: 1.1624x; 1.0044x over previous
import functools

import jax
import jax.numpy as jnp
from jax import lax
from jax.experimental import pallas as pl
from jax.experimental.pallas import tpu as pltpu
from jax.experimental.pallas import tpu_sc as plsc

N = 10000
E = 320000
C = 128
NC = 2
NS = 16
NW = NC * NS
EPW = E // NW
NPAD = 10240
RPS = NPAD // NS
TRASH = NPAD - 1
EPW2 = 10240
CHUNK = 128
NCHUNK = EPW2 // CHUNK
G = 40
NG = NCHUNK // G

_mesh = plsc.VectorSubcoreMesh(core_axis_name="c", subcore_axis_name="s",
                               num_cores=NC, num_subcores=NS)


LASTR = N - (NS - 1) * RPS


def _make_segsum(with_deg: bool):
    out_type = [jax.ShapeDtypeStruct((NC, NPAD, C), jnp.float32)]
    scratch = [
        pltpu.VMEM((G, CHUNK), jnp.int32),
        pltpu.VMEM((G, CHUNK), jnp.int32),
        pltpu.VMEM((CHUNK, C), jnp.float32),
        pltpu.VMEM((CHUNK, C), jnp.float32),
        pltpu.SemaphoreType.DMA,
        pltpu.SemaphoreType.DMA,
        pltpu.SemaphoreType.DMA,
        pltpu.SemaphoreType.DMA,
        pltpu.VMEM_SHARED((NPAD, C), jnp.float32),
    ]
    if with_deg:
        out_type.append(jax.ShapeDtypeStruct((NC, NPAD), jnp.float32))
        scratch += [
            pltpu.VMEM((CHUNK,), jnp.float32),
            pltpu.SemaphoreType.DMA,
            pltpu.VMEM_SHARED((NPAD,), jnp.float32),
        ]

    @functools.partial(pl.kernel, out_type=out_type, mesh=_mesh,
                       scratch_types=scratch)
    def segsum(*refs):
        if with_deg:
            (feat, srcs, dsts, zeros2, zeros1, ones1, aggr_out, deg_out,
             idx_s, idx_d, rows0, rows1, gsem0, gsem1, ssem0, ssem1,
             aggr_sh, ones_v, dsem, deg_sh) = refs
        else:
            (feat, srcs, dsts, zeros2, aggr_out,
             idx_s, idx_d, rows0, rows1, gsem0, gsem1, ssem0, ssem1,
             aggr_sh) = refs
        c = lax.axis_index("c")
        s = lax.axis_index("s")
        wid = s * NC + c
        row0 = s * RPS
        pltpu.sync_copy(zeros2, aggr_sh.at[pl.ds(row0, RPS)])
        if with_deg:
            pltpu.sync_copy(zeros1, deg_sh.at[pl.ds(row0, RPS)])
            pltpu.sync_copy(ones1, ones_v)
        plsc.subcore_barrier()

        def gissue(j, rows, gsem):
            pltpu.async_copy(feat.at[idx_s.at[j]], rows, gsem)

        def gwait(j, rows, gsem):
            pltpu.make_async_copy(feat.at[pl.ds(0, CHUNK)], rows,
                                  gsem).wait()

        def sissue(j, rows, ssem):
            pltpu.async_copy(rows, aggr_sh.at[idx_d.at[j]], ssem, add=True)

        def swait(j, rows, ssem):
            pltpu.make_async_copy(rows, aggr_sh.at[idx_d.at[j]], ssem).wait()

        def dissue(j):
            pltpu.async_copy(ones_v, deg_sh.at[idx_d.at[j]], dsem, add=True)

        def dwait(j):
            pltpu.make_async_copy(feat.at[0], ones_v, dsem).wait()

        def group(g, carry):
            pltpu.sync_copy(srcs.at[wid, pl.ds(g * G, G)], idx_s)
            pltpu.sync_copy(dsts.at[wid, pl.ds(g * G, G)], idx_d)
            gissue(0, rows0, gsem0)
            gissue(1, rows1, gsem1)
            if with_deg:
                dissue(0)

            def body(i, carry2):
                jj = 2 * i
                for b, (rows, gsem) in enumerate(((rows0, gsem0),
                                                  (rows1, gsem1))):
                    j = jj + b
                    gwait(j, rows, gsem)
                    pltpu.sync_copy(rows, aggr_sh.at[idx_d.at[j]], add=True)
                    if with_deg:
                        dwait(j)
                        dissue(j + 1)
                    gissue(j + 2, rows, gsem)
                return carry2

            lax.fori_loop(0, G // 2 - 1, body, 0)
            for b, (rows, gsem) in enumerate(((rows0, gsem0),
                                              (rows1, gsem1))):
                j = G - 2 + b
                gwait(j, rows, gsem)
                pltpu.sync_copy(rows, aggr_sh.at[idx_d.at[j]], add=True)
                if with_deg:
                    dwait(j)
                    if b == 0:
                        dissue(j + 1)
            return carry

        lax.fori_loop(0, NG, group, 0)
        plsc.subcore_barrier()
        pltpu.sync_copy(aggr_sh.at[pl.ds(row0, RPS)],
                        aggr_out.at[c, pl.ds(row0, RPS)])
        if with_deg:
            pltpu.sync_copy(deg_sh.at[pl.ds(row0, RPS)],
                            deg_out.at[c, pl.ds(row0, RPS)])

    return segsum


_segsum_deg = _make_segsum(True)
_segsum = _make_segsum(False)

BLK = 1000
GRID = N // BLK


def _t1_body(aggr_ref, deg_ref, x_ref, wl_ref, wr_ref, b_ref, o_ref):
    aggr = aggr_ref[0] + aggr_ref[1]
    deg = deg_ref[0] + deg_ref[1]
    mean = aggr / jnp.maximum(deg, 1.0)
    h = jnp.dot(mean, wl_ref[...], preferred_element_type=jnp.float32)
    h += jnp.dot(x_ref[...], wr_ref[...], preferred_element_type=jnp.float32)
    h += b_ref[...]
    o_ref[...] = jnp.maximum(h, 0.0)


def _t2_body(aggr_ref, deg_ref, h_ref, wl_ref, wr_ref, b_ref,
             wlin_ref, blin_ref, o_ref):
    aggr = aggr_ref[0] + aggr_ref[1]
    deg = deg_ref[0] + deg_ref[1]
    mean = aggr / jnp.maximum(deg, 1.0)
    h2 = jnp.dot(mean, wl_ref[...], preferred_element_type=jnp.float32)
    h2 += jnp.dot(h_ref[...], wr_ref[...], preferred_element_type=jnp.float32)
    h2 += b_ref[...]
    h2 = jnp.maximum(h2, 0.0)
    o_ref[...] = (jnp.dot(h2, wlin_ref[...], preferred_element_type=jnp.float32)
                  + blin_ref[...])


_W_SPEC = pl.BlockSpec((C, C), lambda i: (0, 0))
_B_SPEC = pl.BlockSpec((1, C), lambda i: (0, 0))
_ROW_SPEC = pl.BlockSpec((BLK, C), lambda i: (i, 0))
_AGGR_SPEC = pl.BlockSpec((NC, BLK, C), lambda i: (0, i, 0))
_DEG_SPEC = pl.BlockSpec((NC, BLK, 1), lambda i: (0, i, 0))

_t1 = pl.pallas_call(
    _t1_body,
    grid=(GRID,),
    in_specs=[_AGGR_SPEC, _DEG_SPEC, _ROW_SPEC, _W_SPEC, _W_SPEC, _B_SPEC],
    out_specs=_ROW_SPEC,
    out_shape=jax.ShapeDtypeStruct((N, C), jnp.float32),
)

_t2 = pl.pallas_call(
    _t2_body,
    grid=(GRID,),
    in_specs=[_AGGR_SPEC, _DEG_SPEC, _ROW_SPEC, _W_SPEC, _W_SPEC, _B_SPEC,
              _W_SPEC, _B_SPEC],
    out_specs=_ROW_SPEC,
    out_shape=jax.ShapeDtypeStruct((N, C), jnp.float32),
)


def kernel(x, edge_index, Wl1, Wr1, b1, Wl2, Wr2, b2, Wlin, blin):
    npad_e = EPW2 - EPW
    src = jnp.concatenate(
        [edge_index[0].astype(jnp.int32).reshape(NW, EPW),
         jnp.broadcast_to(jnp.arange(npad_e, dtype=jnp.int32), (NW, npad_e))],
        axis=1).reshape(NW, NCHUNK, CHUNK)
    dst = jnp.concatenate(
        [edge_index[1].astype(jnp.int32).reshape(NW, EPW),
         jnp.broadcast_to(jnp.arange(N, N + npad_e, dtype=jnp.int32),
                          (NW, npad_e))],
        axis=1).reshape(NW, NCHUNK, CHUNK)
    zeros2 = jnp.zeros((RPS, C), jnp.float32)
    zeros1 = jnp.zeros((RPS,), jnp.float32)
    ones1 = jnp.ones((CHUNK,), jnp.float32)

    aggr1, deg = _segsum_deg(x, src, dst, zeros2, zeros1, ones1)
    deg3 = deg.reshape(NC, NPAD, 1)
    h = _t1(aggr1, deg3, x, Wl1, Wr1, b1.reshape(1, C))
    (aggr2,) = _segsum(h, src, dst, zeros2)
    return _t2(aggr2, deg3, h, Wl2, Wr2, b2.reshape(1, C),
               Wlin, blin.reshape(1, C))
